# baseline (speedup 1.0000x reference)
import functools

import jax
import jax.numpy as jnp
from jax import lax
from jax.experimental import pallas as pl
from jax.experimental.pallas import tpu as pltpu
from jax.experimental.pallas import tpu_sc as plsc

_BATCH = 4
_SEQ_LEN = 2048
_D_MODEL = 1024
_NUM_CORES = 2
_NUM_SUBCORES = 16
_NUM_WORKERS = _NUM_CORES * _NUM_SUBCORES
_ROWS_PER_W = _SEQ_LEN // _NUM_WORKERS


_NUM_CHUNKS = 4
_CHUNK_ROWS = _ROWS_PER_W // _NUM_CHUNKS


@jax.jit
def _positional_lookup(emb):
    mesh = plsc.VectorSubcoreMesh(core_axis_name="c", subcore_axis_name="s")

    @functools.partial(
        pl.kernel,
        out_type=jax.ShapeDtypeStruct((_BATCH, _SEQ_LEN, _D_MODEL), jnp.float32),
        mesh=mesh,
        scratch_types=[
            pltpu.VMEM((_ROWS_PER_W, _D_MODEL), jnp.float32),
            pltpu.SemaphoreType.DMA((_NUM_CHUNKS,)),
            pltpu.SemaphoreType.DMA,
        ],
    )
    def body(emb_hbm, out_hbm, buf, rsems, wsem):
        wid = lax.axis_index("s") * _NUM_CORES + lax.axis_index("c")
        s0 = wid * _ROWS_PER_W
        reads = [
            pltpu.make_async_copy(
                emb_hbm.at[pl.ds(s0 + k * _CHUNK_ROWS, _CHUNK_ROWS)],
                buf.at[pl.ds(k * _CHUNK_ROWS, _CHUNK_ROWS)],
                rsems.at[k],
            )
            for k in range(_NUM_CHUNKS)
        ]
        for r in reads:
            r.start()
        writes = []
        for k in range(_NUM_CHUNKS):
            reads[k].wait()
            for b in range(_BATCH):
                w = pltpu.make_async_copy(
                    buf.at[pl.ds(k * _CHUNK_ROWS, _CHUNK_ROWS)],
                    out_hbm.at[b, pl.ds(s0 + k * _CHUNK_ROWS, _CHUNK_ROWS)],
                    wsem,
                )
                w.start()
                writes.append(w)
        for w in writes:
            w.wait()

    return body(emb)


def kernel(input, emb):
    del input
    return _positional_lookup(emb)


# speedup vs baseline: 1.0233x; 1.0233x over previous
import functools

import jax
import jax.numpy as jnp
from jax import lax
from jax.experimental import pallas as pl
from jax.experimental.pallas import tpu as pltpu
from jax.experimental.pallas import tpu_sc as plsc

_BATCH = 4
_SEQ_LEN = 2048
_D_MODEL = 1024
_NUM_CORES = 2
_NUM_SUBCORES = 16
_NUM_WORKERS = _NUM_CORES * _NUM_SUBCORES
_ROWS_PER_W = _SEQ_LEN // _NUM_WORKERS


_NUM_CHUNKS = 2
_CHUNK_ROWS = _ROWS_PER_W // _NUM_CHUNKS


@jax.jit
def _positional_lookup(emb):
    mesh = plsc.VectorSubcoreMesh(core_axis_name="c", subcore_axis_name="s")

    @functools.partial(
        pl.kernel,
        out_type=jax.ShapeDtypeStruct((_BATCH, _SEQ_LEN, _D_MODEL), jnp.float32),
        mesh=mesh,
        scratch_types=[
            pltpu.VMEM((_ROWS_PER_W, _D_MODEL), jnp.float32),
            pltpu.SemaphoreType.DMA((_NUM_CHUNKS,)),
            pltpu.SemaphoreType.DMA,
        ],
    )
    def body(emb_hbm, out_hbm, buf, rsems, wsem):
        wid = lax.axis_index("s") * _NUM_CORES + lax.axis_index("c")
        s0 = wid * _ROWS_PER_W
        reads = [
            pltpu.make_async_copy(
                emb_hbm.at[pl.ds(s0 + k * _CHUNK_ROWS, _CHUNK_ROWS)],
                buf.at[pl.ds(k * _CHUNK_ROWS, _CHUNK_ROWS)],
                rsems.at[k],
            )
            for k in range(_NUM_CHUNKS)
        ]
        for r in reads:
            r.start()
        writes = []
        for k in range(_NUM_CHUNKS):
            reads[k].wait()
            for b in range(_BATCH):
                w = pltpu.make_async_copy(
                    buf.at[pl.ds(k * _CHUNK_ROWS, _CHUNK_ROWS)],
                    out_hbm.at[b, pl.ds(s0 + k * _CHUNK_ROWS, _CHUNK_ROWS)],
                    wsem,
                )
                w.start()
                writes.append(w)
        for w in writes:
            w.wait()

    return body(emb)


def kernel(input, emb):
    del input
    return _positional_lookup(emb)


# speedup vs baseline: 1.0246x; 1.0013x over previous
import functools

import jax
import jax.numpy as jnp
from jax import lax
from jax.experimental import pallas as pl
from jax.experimental.pallas import tpu as pltpu
from jax.experimental.pallas import tpu_sc as plsc

_BATCH = 4
_SEQ_LEN = 2048
_D_MODEL = 1024
_NUM_CORES = 2
_NUM_SUBCORES = 16
_NUM_WORKERS = _NUM_CORES * _NUM_SUBCORES
_ROWS_PER_W = _SEQ_LEN // _NUM_WORKERS


@jax.jit
def _positional_lookup(emb):
    mesh = plsc.VectorSubcoreMesh(core_axis_name="c", subcore_axis_name="s")

    @functools.partial(
        pl.kernel,
        out_type=jax.ShapeDtypeStruct((_BATCH, _SEQ_LEN, _D_MODEL), jnp.float32),
        mesh=mesh,
        scratch_types=[
            pltpu.VMEM((_ROWS_PER_W, _D_MODEL), jnp.float32),
            pltpu.SemaphoreType.DMA,
        ],
    )
    def body(emb_hbm, out_hbm, buf, sem):
        wid = lax.axis_index("s") * _NUM_CORES + lax.axis_index("c")
        s0 = wid * _ROWS_PER_W
        pltpu.sync_copy(emb_hbm.at[pl.ds(s0, _ROWS_PER_W)], buf)
        copies = [
            pltpu.make_async_copy(
                buf, out_hbm.at[b, pl.ds(s0, _ROWS_PER_W)], sem
            )
            for b in range(_BATCH)
        ]
        for c in copies:
            c.start()
        for c in copies:
            c.wait()

    return body(emb)


def kernel(input, emb):
    del input
    return _positional_lookup(emb)
